# Initial kernel scaffold; baseline (speedup 1.0000x reference)
import jax
import jax.numpy as jnp
from jax.experimental import pallas as pl


def kernel(pos):
    raise NotImplementedError("write your pallas kernel here")


# speedup vs baseline: 1.5820x; 1.5820x over previous
import jax
import jax.numpy as jnp
from jax.experimental import pallas as pl
from jax.experimental.pallas import tpu as pltpu

_B = 65536
_N = 24
_THRESH_SQ = 0.25
_NEG4 = -4.0

_PAIRS = tuple(
    (i, j)
    for i in range(1, _N)
    for j in range(i + 1, _N)
    if not (i == 2 and j == 3)
)

_SB = 64
_LANES = 128
_BS_TOT = _B // _LANES
_GRID = _BS_TOT // _SB


def _collision_body(x_ref, e_ref, c_ref):
    acc_e = jnp.zeros((_SB, _LANES), jnp.float32)
    acc_c = jnp.zeros((_SB, _LANES), jnp.float32)
    for i in range(1, _N):
        xi = x_ref[3 * i]
        yi = x_ref[3 * i + 1]
        zi = x_ref[3 * i + 2]
        for j in range(i + 1, _N):
            if i == 2 and j == 3:
                continue
            dx = xi - x_ref[3 * j]
            dy = yi - x_ref[3 * j + 1]
            dz = zi - x_ref[3 * j + 2]
            sq = dx * dx + dy * dy + dz * dz
            sel = sq < _THRESH_SQ
            e = jnp.exp(sq * _NEG4)
            acc_e = acc_e + jnp.where(sel, e, 0.0)
            acc_c = acc_c + jnp.where(sel, 1.0, 0.0)
    e_ref[...] = jnp.sum(acc_e, axis=0).reshape(1, 1, _LANES)
    c_ref[...] = jnp.sum(acc_c, axis=0).reshape(1, 1, _LANES)


def kernel(pos):
    x2 = pos.reshape(_B, 3 * _N)
    xt = x2.T.reshape(3 * _N, _BS_TOT, _LANES)

    e_part, c_part = pl.pallas_call(
        _collision_body,
        grid=(_GRID,),
        in_specs=[
            pl.BlockSpec((3 * _N, _SB, _LANES), lambda g: (0, g, 0)),
        ],
        out_specs=[
            pl.BlockSpec((1, 1, _LANES), lambda g: (g, 0, 0)),
            pl.BlockSpec((1, 1, _LANES), lambda g: (g, 0, 0)),
        ],
        out_shape=[
            jax.ShapeDtypeStruct((_GRID, 1, _LANES), jnp.float32),
            jax.ShapeDtypeStruct((_GRID, 1, _LANES), jnp.float32),
        ],
        compiler_params=pltpu.CompilerParams(
            dimension_semantics=("parallel",),
        ),
    )(xt)

    se = jnp.sum(e_part)
    cnt = jnp.sum(c_part)
    total = jnp.where(cnt > 0, se / jnp.maximum(cnt, 1.0), 0.0)
    return total + 1e-6
